# baseline (speedup 1.0000x reference)
import functools

import jax
import jax.numpy as jnp
from jax import lax
from jax.experimental import pallas as pl
from jax.experimental.pallas import tpu as pltpu
from jax.experimental.pallas import tpu_sc as plsc

_N = 10000
_E = 320000
_D = 128
_DH = _D // 2

_NC = 2
_NS = 16
_B = 125
_CH = 160
_EPT = _B * _CH
_NB = 4
_LA = 2
_NA = _N + 8
_RPT = 624
_RLAST = _N - (_NS - 1) * _RPT


def _make_segsum():
    mesh = plsc.VectorSubcoreMesh(core_axis_name="c", subcore_axis_name="s")

    @functools.partial(
        pl.kernel,
        out_type=jax.ShapeDtypeStruct((_N, _D), jnp.float32),
        mesh=mesh,
        scratch_types=[
            pltpu.VMEM((_CH, _B), jnp.int32),
            pltpu.VMEM((_CH, _B), jnp.int32),
            [pltpu.VMEM((_B, _DH), jnp.float32) for _ in range(_NB)],
            pltpu.VMEM_SHARED((_NA, _DH), jnp.float32),
            [pltpu.SemaphoreType.DMA for _ in range(_NB)],
            [pltpu.SemaphoreType.DMA for _ in range(_NB)],
            pltpu.SemaphoreType.DMA,
        ],
        compiler_params=pltpu.CompilerParams(use_tc_tiling_on_sc=False),
    )
    def segsum(feat_hbm, edges_hbm, zeros_hbm, out_hbm, sidx_v, didx_v,
               rows, acc_sh, gsem, ssem, psem):
        c = lax.axis_index("c")
        s = lax.axis_index("s")
        tab = feat_hbm.at[c]
        icp0 = pltpu.async_copy(edges_hbm.at[0, s], sidx_v, psem)
        icp1 = pltpu.async_copy(edges_hbm.at[1, s], didx_v, psem)

        @pl.when(s < _NS - 1)
        def _():
            pltpu.sync_copy(zeros_hbm.at[pl.ds(0, _RPT)],
                            acc_sh.at[pl.ds(s * _RPT, _RPT)])

        @pl.when(s == _NS - 1)
        def _():
            pltpu.sync_copy(zeros_hbm, acc_sh.at[pl.ds(s * _RPT, _RLAST)])

        icp0.wait()
        icp1.wait()
        plsc.subcore_barrier()

        def start_gather(i, b):
            return pltpu.async_copy(tab.at[sidx_v.at[i]], rows[b], gsem[b])

        def wait_gather(i, b):
            pltpu.make_async_copy(tab.at[sidx_v.at[i]], rows[b],
                                  gsem[b]).wait()

        def start_scatter(i, b):
            return pltpu.async_copy(rows[b], acc_sh.at[didx_v.at[i]], ssem[b],
                                    add=True)

        def wait_scatter(i, b):
            pltpu.make_async_copy(rows[b], acc_sh.at[didx_v.at[i]],
                                  ssem[b]).wait()

        for k in range(_LA):
            start_gather(k, k)

        def body(j, carry):
            for b in range(_NB):
                i = _NB * j + b
                wait_gather(i, b)
                start_scatter(i, b)
                nxt = (b + _LA) % _NB

                @pl.when(i + _LA < _CH)
                def _():
                    @pl.when(i >= _LA)
                    def _():
                        wait_scatter(i - _LA, nxt)
                    start_gather(i + _LA, nxt)
            return carry

        lax.fori_loop(0, _CH // _NB, body, 0)
        for k in range(2 * _LA):
            i = _CH - 2 * _LA + k
            wait_scatter(i, i % _NB)
        plsc.subcore_barrier()

        @pl.when(s < _NS - 1)
        def _():
            pltpu.sync_copy(acc_sh.at[pl.ds(s * _RPT, _RPT)],
                            out_hbm.at[pl.ds(s * _RPT, _RPT),
                                       pl.ds(c * _DH, _DH)])

        @pl.when(s == _NS - 1)
        def _():
            pltpu.sync_copy(acc_sh.at[pl.ds(s * _RPT, _RLAST)],
                            out_hbm.at[pl.ds(s * _RPT, _RLAST),
                                       pl.ds(c * _DH, _DH)])

    return segsum


_segsum = _make_segsum()

_BN = 5000
_GRID = _N // _BN


def _mlp1_body(f_ref, a_ref, w_ref, b_ref, o_ref):
    x = f_ref[...] + a_ref[...]
    y = lax.dot_general(x, w_ref[...], (((1,), (1,)), ((), ())),
                        preferred_element_type=jnp.float32,
                        precision=lax.Precision.HIGHEST)
    y = jnp.maximum(y + b_ref[...], 0.0)
    o_ref[0] = y[:, :_DH]
    o_ref[1] = y[:, _DH:]


def _mlp2_body(h_ref, a_ref, w_ref, b_ref, o_ref):
    x = jnp.concatenate([h_ref[0], h_ref[1]], axis=1) + a_ref[...]
    y = lax.dot_general(x, w_ref[...], (((1,), (1,)), ((), ())),
                        preferred_element_type=jnp.float32,
                        precision=lax.Precision.HIGHEST)
    y = y + b_ref[...]
    m = jnp.max(y, axis=1, keepdims=True)
    lse = m + jnp.log(jnp.sum(jnp.exp(y - m), axis=1, keepdims=True))
    o_ref[...] = y - lse


_SPLIT_SPEC = pl.BlockSpec((_NC, _BN, _DH), lambda i: (0, i, 0))
_FULL_SPEC = pl.BlockSpec((_BN, _D), lambda i: (i, 0))
_W_SPEC = pl.BlockSpec((_D, _D), lambda i: (0, 0))
_B_SPEC = pl.BlockSpec((1, _D), lambda i: (0, 0))


def _mlp1(feature, aggs, W, b):
    return pl.pallas_call(
        _mlp1_body,
        grid=(_GRID,),
        in_specs=[_FULL_SPEC, _FULL_SPEC, _W_SPEC, _B_SPEC],
        out_specs=_SPLIT_SPEC,
        out_shape=jax.ShapeDtypeStruct((_NC, _N, _DH), jnp.float32),
    )(feature, aggs, W, b.reshape(1, _D))


def _mlp2(h, aggs, W, b):
    return pl.pallas_call(
        _mlp2_body,
        grid=(_GRID,),
        in_specs=[_SPLIT_SPEC, _FULL_SPEC, _W_SPEC, _B_SPEC],
        out_specs=_FULL_SPEC,
        out_shape=jax.ShapeDtypeStruct((_N, _D), jnp.float32),
    )(h, aggs, W, b.reshape(1, _D))


def kernel(feature, edge_index, W1, b1, W2, b2):
    edges = edge_index.reshape(2, _NS, _CH, _B)
    zeros = jnp.zeros((_RLAST, _DH), jnp.float32)
    feat2 = jnp.stack([feature[:, :_DH], feature[:, _DH:]])

    agg1 = _segsum(feat2, edges, zeros)
    h2 = _mlp1(feature, agg1, W1, b1)
    agg2 = _segsum(h2, edges, zeros)
    return _mlp2(h2, agg2, W2, b2)


# speedup vs baseline: 1.0130x; 1.0130x over previous
import functools

import jax
import jax.numpy as jnp
from jax import lax
from jax.experimental import pallas as pl
from jax.experimental.pallas import tpu as pltpu
from jax.experimental.pallas import tpu_sc as plsc

_N = 10000
_E = 320000
_D = 128
_DH = _D // 2

_NC = 2
_NS = 16
_B = 125
_CH = 160
_EPT = _B * _CH
_NB = 4
_LA = 2
_NA = _N + 8
_RPT = 624
_RLAST = _N - (_NS - 1) * _RPT


def _make_segsum():
    mesh = plsc.VectorSubcoreMesh(core_axis_name="c", subcore_axis_name="s")

    @functools.partial(
        pl.kernel,
        out_type=jax.ShapeDtypeStruct((_N, _D), jnp.float32),
        mesh=mesh,
        scratch_types=[
            pltpu.VMEM((_CH, _B), jnp.int32),
            pltpu.VMEM((_CH, _B), jnp.int32),
            [pltpu.VMEM((_B, _DH), jnp.float32) for _ in range(_NB)],
            pltpu.VMEM_SHARED((_NA, _DH), jnp.float32),
            [pltpu.SemaphoreType.DMA for _ in range(_NB)],
            [pltpu.SemaphoreType.DMA for _ in range(_NB)],
            pltpu.SemaphoreType.DMA,
        ],
        compiler_params=pltpu.CompilerParams(use_tc_tiling_on_sc=False),
    )
    def segsum(feat_hbm, edges_hbm, zeros_hbm, out_hbm, sidx_v, didx_v,
               rows, acc_sh, gsem, ssem, psem):
        c = lax.axis_index("c")
        s = lax.axis_index("s")
        tab = feat_hbm.at[c]
        icp0 = pltpu.async_copy(edges_hbm.at[0, s], sidx_v, psem)
        icp1 = pltpu.async_copy(edges_hbm.at[1, s], didx_v, psem)

        @pl.when(s < _NS - 1)
        def _():
            pltpu.sync_copy(zeros_hbm.at[pl.ds(0, _RPT)],
                            acc_sh.at[pl.ds(s * _RPT, _RPT)])

        @pl.when(s == _NS - 1)
        def _():
            pltpu.sync_copy(zeros_hbm, acc_sh.at[pl.ds(s * _RPT, _RLAST)])

        icp0.wait()
        icp1.wait()
        plsc.subcore_barrier()

        def start_gather(i, b):
            return pltpu.async_copy(tab.at[sidx_v.at[i]], rows[b], gsem[b])

        def wait_gather(i, b):
            pltpu.make_async_copy(tab.at[sidx_v.at[i]], rows[b],
                                  gsem[b]).wait()

        def start_scatter(i, b):
            return pltpu.async_copy(rows[b], acc_sh.at[didx_v.at[i]], ssem[b],
                                    add=True)

        def wait_scatter(i, b):
            pltpu.make_async_copy(rows[b], acc_sh.at[didx_v.at[i]],
                                  ssem[b]).wait()

        for k in range(_LA):
            start_gather(k, k)

        def body(j, carry):
            for b in range(_NB):
                i = _NB * j + b
                wait_gather(i, b)
                start_scatter(i, b)
                nxt = (b + _LA) % _NB

                @pl.when(i + _LA < _CH)
                def _():
                    @pl.when(i >= _LA)
                    def _():
                        wait_scatter(i - _LA, nxt)
                    start_gather(i + _LA, nxt)
            return carry

        lax.fori_loop(0, _CH // _NB, body, 0)
        for k in range(2 * _LA):
            i = _CH - 2 * _LA + k
            wait_scatter(i, i % _NB)
        plsc.subcore_barrier()

        @pl.when(s < _NS - 1)
        def _():
            pltpu.sync_copy(acc_sh.at[pl.ds(s * _RPT, _RPT)],
                            out_hbm.at[pl.ds(s * _RPT, _RPT),
                                       pl.ds(c * _DH, _DH)])

        @pl.when(s == _NS - 1)
        def _():
            pltpu.sync_copy(acc_sh.at[pl.ds(s * _RPT, _RLAST)],
                            out_hbm.at[pl.ds(s * _RPT, _RLAST),
                                       pl.ds(c * _DH, _DH)])

    return segsum


_segsum = _make_segsum()

_BN = 2000
_GRID = _N // _BN


def _mlp1_body(f_ref, a_ref, w_ref, b_ref, o_ref):
    x = f_ref[...] + a_ref[...]
    y = lax.dot_general(x, w_ref[...], (((1,), (1,)), ((), ())),
                        preferred_element_type=jnp.float32,
                        precision=lax.Precision.HIGHEST)
    y = jnp.maximum(y + b_ref[...], 0.0)
    o_ref[0] = y[:, :_DH]
    o_ref[1] = y[:, _DH:]


def _mlp2_body(h_ref, a_ref, w_ref, b_ref, o_ref):
    x = jnp.concatenate([h_ref[0], h_ref[1]], axis=1) + a_ref[...]
    y = lax.dot_general(x, w_ref[...], (((1,), (1,)), ((), ())),
                        preferred_element_type=jnp.float32,
                        precision=lax.Precision.HIGHEST)
    y = y + b_ref[...]
    m = jnp.max(y, axis=1, keepdims=True)
    lse = m + jnp.log(jnp.sum(jnp.exp(y - m), axis=1, keepdims=True))
    o_ref[...] = y - lse


_SPLIT_SPEC = pl.BlockSpec((_NC, _BN, _DH), lambda i: (0, i, 0))
_FULL_SPEC = pl.BlockSpec((_BN, _D), lambda i: (i, 0))
_W_SPEC = pl.BlockSpec((_D, _D), lambda i: (0, 0))
_B_SPEC = pl.BlockSpec((1, _D), lambda i: (0, 0))


def _mlp1(feature, aggs, W, b):
    return pl.pallas_call(
        _mlp1_body,
        grid=(_GRID,),
        in_specs=[_FULL_SPEC, _FULL_SPEC, _W_SPEC, _B_SPEC],
        out_specs=_SPLIT_SPEC,
        out_shape=jax.ShapeDtypeStruct((_NC, _N, _DH), jnp.float32),
    )(feature, aggs, W, b.reshape(1, _D))


def _mlp2(h, aggs, W, b):
    return pl.pallas_call(
        _mlp2_body,
        grid=(_GRID,),
        in_specs=[_SPLIT_SPEC, _FULL_SPEC, _W_SPEC, _B_SPEC],
        out_specs=_FULL_SPEC,
        out_shape=jax.ShapeDtypeStruct((_N, _D), jnp.float32),
    )(h, aggs, W, b.reshape(1, _D))


def kernel(feature, edge_index, W1, b1, W2, b2):
    edges = edge_index.reshape(2, _NS, _CH, _B)
    zeros = jnp.zeros((_RLAST, _DH), jnp.float32)
    feat2 = jnp.stack([feature[:, :_DH], feature[:, _DH:]])

    agg1 = _segsum(feat2, edges, zeros)
    h2 = _mlp1(feature, agg1, W1, b1)
    agg2 = _segsum(h2, edges, zeros)
    return _mlp2(h2, agg2, W2, b2)
